# baseline (speedup 1.0000x reference)
import functools

import jax
import jax.numpy as jnp
from jax import lax
from jax.experimental import pallas as pl
from jax.experimental.pallas import tpu as pltpu
from jax.experimental.pallas import tpu_sc as plsc

_NUM_ROIS = 16384
_DATA = 360
_L = 16
_NFULL = _DATA // _L
_TAIL_OFF = _DATA - _L
_NW = 32
_ROWS_PER_W = _NUM_ROIS // _NW
_RBLK = 32
_NBLK = _ROWS_PER_W // _RBLK
_NW24 = 24
_SUB = 8
_OUT_BYTES = _RBLK * _DATA * 4
_TCB = 256


def _tc_flags_body(y_ref, flags_ref):
    yb = y_ref[0]
    col = lax.broadcasted_iota(jnp.int32, yb.shape, 1)
    valid = jnp.logical_and(col >= _DATA, col < 2 * _DATA)
    vals = jnp.where(valid, jnp.abs(yb), 1.0)
    flags_ref[...] = jnp.min(vals, axis=1)


def _process_row(r_src, r_dst, mask_v, pred_v, out_v, zbuf):
    iota = lax.iota(jnp.int32, _L)
    r_splat = jnp.full((_L,), r_dst, jnp.int32)

    def chunk(c, nz_carry):
        off = c * _L
        w = plsc.load_gather(
            mask_v, [jnp.full((_L,), r_src, jnp.int32),
                     jnp.full((_L,), c, jnp.int32)])
        p = pred_v[r_src, pl.ds(off, _L)]
        nz = ((w >> iota) & 1) == 1
        cum = plsc.cumsum(nz.astype(jnp.int32))
        plsc.store_scatter(out_v, [r_splat, nz_carry + cum - 1], p, mask=nz)
        pos_z = (off - nz_carry) + (iota + 1 - cum) - 1
        plsc.store_scatter(zbuf, [pos_z], p, mask=jnp.logical_not(nz))
        return nz_carry + plsc.all_reduce_population_count(nz)

    nz_carry = lax.fori_loop(
        0, _NFULL, chunk, jnp.zeros((_L,), jnp.int32))

    w22 = plsc.load_gather(
        mask_v, [jnp.full((_L,), r_src, jnp.int32),
                 jnp.full((_L,), _NFULL, jnp.int32)])
    p = pred_v[r_src, pl.ds(_TAIL_OFF, _L)]
    valid = iota >= (_L - (_DATA - _NFULL * _L))
    nz_bit = (w22 >> jnp.maximum(iota - 8, 0)) & 1
    nz = jnp.logical_and(nz_bit == 1, valid)
    vcnt = jnp.maximum(iota - 7, 0)
    cum = plsc.cumsum(nz.astype(jnp.int32))
    plsc.store_scatter(out_v, [r_splat, nz_carry + cum - 1], p, mask=nz)
    zm = jnp.logical_and(valid, jnp.logical_not(nz))
    pos_z = (_NFULL * _L - nz_carry) + (vcnt - cum) - 1
    plsc.store_scatter(zbuf, [pos_z], p, mask=zm)
    nz_carry = nz_carry + plsc.all_reduce_population_count(nz)

    zc = _DATA - nz_carry
    zc_s = jnp.max(zc)

    @pl.when(zc_s > 0)
    def _append_zeros():
        def append(c, carry):
            off = c * _L
            zv = zbuf[pl.ds(off, _L)]
            i_vec = off + iota
            pos = jnp.minimum(nz_carry + i_vec, _DATA - 1)
            plsc.store_scatter(out_v, [r_splat, pos], zv, mask=i_vec < zc)
            return carry

        lax.fori_loop(0, _NFULL + 1, append, 0)

    return 0


def _sc_body(pred_hbm, mask_hbm, out_hbm, dummy_hbm,
             mw_v, p_v, o_v, zbuf, smw, sip, sp, so):
    wid = lax.axis_index("c") * 16 + lax.axis_index("s")
    w0 = wid * _ROWS_PER_W
    iota = lax.iota(jnp.int32, _L)

    def base_of(b):
        return w0 + b * _RBLK

    def start_p(b, j):
        base = base_of(b)
        pltpu.async_copy(
            pred_hbm.at[0, pl.ds(base, _RBLK), :], p_v[j], sip[j])
        pltpu.async_copy(
            mask_hbm.at[pl.ds(base, _RBLK), :], mw_v[j], smw[j])

    def wait_p(j):
        pltpu.make_async_copy(
            pred_hbm.at[0, pl.ds(0, _RBLK), :], p_v[j], sip[j]).wait()
        pltpu.make_async_copy(
            mask_hbm.at[pl.ds(0, _RBLK), :], mw_v[j], smw[j]).wait()

    def wait_out_block():
        pltpu.make_async_copy(
            p_v[0], out_hbm.at[0, pl.ds(0, _RBLK), :], so).wait()

    _e0 = jnp.full((_L,), 0xFFFF, jnp.int32)
    _e1 = jnp.where(iota < 14, 0xFFFF, jnp.where(iota == 14, 0xFF, 0))

    def process(b, j):
        base = base_of(b)

        @pl.when(b >= 3)
        def _drain_prev0():
            pass

        @pl.when(b >= 3)
        def _drain_prev():
            wait_out_block()

        @pl.when(b + 1 < _NBLK)
        def _prefetch():
            start_p(b + 1, (j + 1) % 4)

        wait_p(j)

        def chk(r, acc):
            d0 = mw_v[j][r, pl.ds(0, _L)] ^ _e0
            d1 = mw_v[j][r, pl.ds(_NW24 - _L, _L)] ^ _e1
            return acc | d0 | d1

        acc = lax.fori_loop(0, _RBLK, chk, jnp.zeros((_L,), jnp.int32))
        clean = jnp.max(acc) == 0

        @pl.when(clean)
        def _fast():
            pltpu.async_copy(p_v[j], out_hbm.at[0, pl.ds(base, _RBLK), :],
                             so)

        @pl.when(jnp.logical_not(clean))
        def _slow():
            def sub(sb, carry):
                lax.fori_loop(
                    0, _SUB,
                    lambda rr, cc: _process_row(
                        sb * _SUB + rr, rr, mw_v[j], p_v[j], o_v, zbuf),
                    0)
                oc = pltpu.make_async_copy(
                    o_v,
                    out_hbm.at[0, pl.ds(base + sb * _SUB, _SUB), :], sp)
                oc.start()
                oc.wait()
                return carry

            lax.fori_loop(0, _RBLK // _SUB, sub, 0)
            pltpu.async_copy(p_v[j], dummy_hbm, so)

    start_p(0, 0)

    def body(i, carry):
        for jj in range(4):
            process(4 * i + jj, jj)
        return carry

    lax.fori_loop(0, _NBLK // 4, body, 0)
    for _ in range(3):
        wait_out_block()


@jax.jit
def kernel(pred, y):
    mask2 = y[0, :, _DATA:2 * _DATA]
    nw = _DATA // _L + 1
    mw23 = jnp.zeros((_NUM_ROIS, nw), jnp.int32)
    for k in range(_L):
        col = (mask2[:, k::_L] != 0.0).astype(jnp.int32) << k
        mw23 = mw23.at[:, :col.shape[1]].add(col)
    mwords = jnp.pad(mw23, ((0, 0), (0, _NW24 - nw)))

    run = pl.kernel(
        _sc_body,
        out_type=(jax.ShapeDtypeStruct((1, _NUM_ROIS, _DATA), jnp.float32),
                  jax.ShapeDtypeStruct((_RBLK, _DATA), jnp.float32)),
        mesh=plsc.VectorSubcoreMesh(core_axis_name="c", subcore_axis_name="s"),
        compiler_params=pltpu.CompilerParams(needs_layout_passes=False),
        scratch_types=[
            [pltpu.VMEM((_RBLK, _NW24), jnp.int32) for _ in range(4)],
            [pltpu.VMEM((_RBLK, _DATA), jnp.float32) for _ in range(4)],
            pltpu.VMEM((_SUB, _DATA), jnp.float32),
            pltpu.VMEM((_NFULL * _L + _L * 2,), jnp.float32),
            [pltpu.SemaphoreType.DMA for _ in range(4)],
            [pltpu.SemaphoreType.DMA for _ in range(4)],
            pltpu.SemaphoreType.DMA,
            pltpu.SemaphoreType.DMA,
        ],
    )
    return run(pred, mwords)[0]


# speedup vs baseline: 10.1980x; 10.1980x over previous
import jax
import jax.numpy as jnp
from jax import lax
from jax.experimental import pallas as pl
from jax.experimental.pallas import tpu as pltpu
from jax.experimental.pallas import tpu_sc as plsc

_NUM_ROIS = 16384
_DATA = 360
_L = 16
_NFULL = _DATA // _L
_TAIL_OFF = _DATA - _L
_NW = 32
_ROWS_PER_W = _NUM_ROIS // _NW
_RBLK = 32
_NBLK = _ROWS_PER_W // _RBLK
_NW24 = 24
_SUB = 8
_OUT_BYTES = _RBLK * _DATA * 4
_TCB = 256


def _tc_flags_body(y_ref, flags_ref):
    yb = y_ref[0]
    col = lax.broadcasted_iota(jnp.int32, yb.shape, 1)
    valid = jnp.logical_and(col >= _DATA, col < 2 * _DATA)
    vals = jnp.where(valid, jnp.abs(yb), 1.0)
    flags_ref[...] = jnp.min(vals, axis=1)


def _process_row(r_src, r_dst, mask_v, pred_v, out_v, zbuf):
    iota = lax.iota(jnp.int32, _L)
    r_splat = jnp.full((_L,), r_dst, jnp.int32)

    def chunk(c, nz_carry):
        off = c * _L
        w = plsc.load_gather(
            mask_v, [jnp.full((_L,), r_src, jnp.int32),
                     jnp.full((_L,), c, jnp.int32)])
        p = pred_v[r_src, pl.ds(off, _L)]
        nz = ((w >> iota) & 1) == 1
        cum = plsc.cumsum(nz.astype(jnp.int32))
        plsc.store_scatter(out_v, [r_splat, nz_carry + cum - 1], p, mask=nz)
        pos_z = (off - nz_carry) + (iota + 1 - cum) - 1
        plsc.store_scatter(zbuf, [pos_z], p, mask=jnp.logical_not(nz))
        return nz_carry + plsc.all_reduce_population_count(nz)

    nz_carry = lax.fori_loop(
        0, _NFULL, chunk, jnp.zeros((_L,), jnp.int32))

    w22 = plsc.load_gather(
        mask_v, [jnp.full((_L,), r_src, jnp.int32),
                 jnp.full((_L,), _NFULL, jnp.int32)])
    p = pred_v[r_src, pl.ds(_TAIL_OFF, _L)]
    valid = iota >= (_L - (_DATA - _NFULL * _L))
    nz_bit = (w22 >> jnp.maximum(iota - 8, 0)) & 1
    nz = jnp.logical_and(nz_bit == 1, valid)
    vcnt = jnp.maximum(iota - 7, 0)
    cum = plsc.cumsum(nz.astype(jnp.int32))
    plsc.store_scatter(out_v, [r_splat, nz_carry + cum - 1], p, mask=nz)
    zm = jnp.logical_and(valid, jnp.logical_not(nz))
    pos_z = (_NFULL * _L - nz_carry) + (vcnt - cum) - 1
    plsc.store_scatter(zbuf, [pos_z], p, mask=zm)
    nz_carry = nz_carry + plsc.all_reduce_population_count(nz)

    zc = _DATA - nz_carry
    zc_s = jnp.max(zc)

    @pl.when(zc_s > 0)
    def _append_zeros():
        def append(c, carry):
            off = c * _L
            zv = zbuf[pl.ds(off, _L)]
            i_vec = off + iota
            pos = jnp.minimum(nz_carry + i_vec, _DATA - 1)
            plsc.store_scatter(out_v, [r_splat, pos], zv, mask=i_vec < zc)
            return carry

        lax.fori_loop(0, _NFULL + 1, append, 0)

    return 0


def _sc_body(pred_hbm, mask_hbm, out_hbm, dummy_hbm,
             mw_v, p_v, o_v, zbuf, smw, sip, sp, so):
    wid = lax.axis_index("c") * 16 + lax.axis_index("s")
    w0 = wid * _ROWS_PER_W
    iota = lax.iota(jnp.int32, _L)

    def base_of(b):
        return w0 + b * _RBLK

    def start_p(b, j):
        base = base_of(b)
        pltpu.async_copy(
            pred_hbm.at[0, pl.ds(base, _RBLK), :], p_v[j], sip[j])
        pltpu.async_copy(
            mask_hbm.at[pl.ds(base, _RBLK), :], mw_v[j], smw[j])

    def wait_p(j):
        pltpu.make_async_copy(
            pred_hbm.at[0, pl.ds(0, _RBLK), :], p_v[j], sip[j]).wait()
        pltpu.make_async_copy(
            mask_hbm.at[pl.ds(0, _RBLK), :], mw_v[j], smw[j]).wait()

    def wait_out_block():
        pltpu.make_async_copy(
            p_v[0], out_hbm.at[0, pl.ds(0, _RBLK), :], so).wait()

    _e0 = jnp.full((_L,), 0xFFFF, jnp.int32)
    _e1 = jnp.where(iota < 14, 0xFFFF, jnp.where(iota == 14, 0xFF, 0))

    def process(b, j):
        base = base_of(b)


        @pl.when(b >= 3)
        def _drain_prev():
            wait_out_block()

        @pl.when(b + 1 < _NBLK)
        def _prefetch():
            start_p(b + 1, (j + 1) % 4)

        wait_p(j)

        def chk(r, acc):
            d0 = mw_v[j][r, pl.ds(0, _L)] ^ _e0
            d1 = mw_v[j][r, pl.ds(_NW24 - _L, _L)] ^ _e1
            return acc | d0 | d1

        acc = lax.fori_loop(0, _RBLK, chk, jnp.zeros((_L,), jnp.int32))
        clean = jnp.max(acc) == 0

        @pl.when(clean)
        def _fast():
            pltpu.async_copy(p_v[j], out_hbm.at[0, pl.ds(base, _RBLK), :],
                             so)

        @pl.when(jnp.logical_not(clean))
        def _slow():
            def sub(sb, carry):
                lax.fori_loop(
                    0, _SUB,
                    lambda rr, cc: _process_row(
                        sb * _SUB + rr, rr, mw_v[j], p_v[j], o_v, zbuf),
                    0)
                oc = pltpu.make_async_copy(
                    o_v,
                    out_hbm.at[0, pl.ds(base + sb * _SUB, _SUB), :], sp)
                oc.start()
                oc.wait()
                return carry

            lax.fori_loop(0, _RBLK // _SUB, sub, 0)
            pltpu.async_copy(p_v[j], dummy_hbm, so)

    start_p(0, 0)

    def body(i, carry):
        for jj in range(4):
            process(4 * i + jj, jj)
        return carry

    lax.fori_loop(0, _NBLK // 4, body, 0)
    for _ in range(3):
        wait_out_block()


@jax.jit
def kernel(pred, y):
    mask2 = y[0, :, _DATA:2 * _DATA]
    nzb = (mask2 != 0.0).astype(jnp.int32)
    nzb = jnp.pad(nzb, ((0, 0), (0, _NW24 * _L - _DATA)))
    weights = (1 << jnp.arange(_L, dtype=jnp.int32))
    mwords = jnp.einsum("rck,k->rc",
                        nzb.reshape(_NUM_ROIS, _NW24, _L), weights,
                        preferred_element_type=jnp.int32)

    run = pl.kernel(
        _sc_body,
        out_type=(jax.ShapeDtypeStruct((1, _NUM_ROIS, _DATA), jnp.float32),
                  jax.ShapeDtypeStruct((_RBLK, _DATA), jnp.float32)),
        mesh=plsc.VectorSubcoreMesh(core_axis_name="c", subcore_axis_name="s"),
        compiler_params=pltpu.CompilerParams(needs_layout_passes=False),
        scratch_types=[
            [pltpu.VMEM((_RBLK, _NW24), jnp.int32) for _ in range(4)],
            [pltpu.VMEM((_RBLK, _DATA), jnp.float32) for _ in range(4)],
            pltpu.VMEM((_SUB, _DATA), jnp.float32),
            pltpu.VMEM((_NFULL * _L + _L * 2,), jnp.float32),
            [pltpu.SemaphoreType.DMA for _ in range(4)],
            [pltpu.SemaphoreType.DMA for _ in range(4)],
            pltpu.SemaphoreType.DMA,
            pltpu.SemaphoreType.DMA,
        ],
    )
    return run(pred, mwords)[0]
